# baseline (speedup 1.0000x reference)
import functools

import jax
import jax.numpy as jnp
from jax import lax
from jax.experimental import pallas as pl
from jax.experimental.pallas import tpu as pltpu
from jax.experimental.pallas import tpu_sc as plsc

_N = 50000
_E = 800000
_EMB = 64
_HID = 128
_EDGE = 8

_NC = 2
_NS = 16
_NW = _NC * _NS

_EA = 409600
_EB = _E - _EA

_GCHUNK = 512

_HALF = _N // 2
_QUART = _HALF // 2
_ODD_BASE = 12800
_PAD_BASE = 25300
_ACC_ROWS = 25600

_BLK_N = 5000
_BLK_E = 3200


def _sc_mesh():
    return plsc.VectorSubcoreMesh(
        core_axis_name="c", subcore_axis_name="s",
        num_cores=_NC, num_subcores=_NS)


def _gather(h64, src, dst, e0, n):
    per_w = n // _NW
    n_full = per_w // _GCHUNK
    tail = per_w - n_full * _GCHUNK

    @functools.partial(
        pl.kernel,
        out_type=jax.ShapeDtypeStruct((n, 2 * _EMB), jnp.float32),
        mesh=_sc_mesh(),
        scratch_types=[
            pltpu.VMEM((_GCHUNK,), jnp.int32),
            pltpu.VMEM((_GCHUNK,), jnp.int32),
            pltpu.VMEM((_GCHUNK, _EMB), jnp.float32),
            pltpu.VMEM((_GCHUNK, _EMB), jnp.float32),
            pltpu.SemaphoreType.DMA,
            pltpu.SemaphoreType.DMA,
            pltpu.SemaphoreType.DMA,
        ],
        compiler_params=pltpu.CompilerParams(use_tc_tiling_on_sc=False),
    )
    def k(h_hbm, src_hbm, dst_hbm, cat_hbm,
          idx_a, idx_b, rows_a, rows_b, sem_i, sem_g, sem_o):
        c = lax.axis_index("c")
        s = lax.axis_index("s")
        wid = s * _NC + c
        base = wid * per_w

        full_slices = [(j * 128, 128) for j in range(_GCHUNK // 128)]
        tail_slices = [(j * 128, 128) for j in range(tail // 128)]
        if tail % 128:
            tail_slices.append((tail - tail % 128, tail % 128))

        def out_copy(rows_v, off, col, m):
            return (rows_v.at[pl.ds(0, m)],
                    cat_hbm.at[pl.ds(off - e0, m), pl.ds(col, _EMB)])

        def step(i, carry):
            off = e0 + base + i * _GCHUNK

            @pl.when(i > 0)
            def _():
                sa, da = out_copy(rows_a, off, 0, _GCHUNK)
                pltpu.make_async_copy(sa, da, sem_o).wait()
                sb, db = out_copy(rows_b, off, _EMB, _GCHUNK)
                pltpu.make_async_copy(sb, db, sem_o).wait()

            ca = pltpu.async_copy(dst_hbm.at[pl.ds(off, _GCHUNK)],
                                  idx_a, sem_i)
            cb = pltpu.async_copy(src_hbm.at[pl.ds(off, _GCHUNK)],
                                  idx_b, sem_i)
            ca.wait()
            cb.wait()
            cps = [pltpu.async_copy(
                h_hbm.at[idx_a.at[pl.ds(st, ln)]],
                rows_a.at[pl.ds(st, ln)], sem_g)
                for (st, ln) in full_slices]
            cps += [pltpu.async_copy(
                h_hbm.at[idx_b.at[pl.ds(st, ln)]],
                rows_b.at[pl.ds(st, ln)], sem_g)
                for (st, ln) in full_slices]
            for cp in cps:
                cp.wait()
            sa, da = out_copy(rows_a, off, 0, _GCHUNK)
            pltpu.async_copy(sa, da, sem_o)
            sb, db = out_copy(rows_b, off, _EMB, _GCHUNK)
            pltpu.async_copy(sb, db, sem_o)
            return carry

        lax.fori_loop(0, n_full, step, 0)
        sa, da = out_copy(rows_a, e0 + base, 0, _GCHUNK)
        pltpu.make_async_copy(sa, da, sem_o).wait()
        sb, db = out_copy(rows_b, e0 + base, _EMB, _GCHUNK)
        pltpu.make_async_copy(sb, db, sem_o).wait()

        if tail:
            toff = e0 + base + n_full * _GCHUNK

            def one(idx_hbm, idx_v, rows_v, col):
                pltpu.sync_copy(idx_hbm.at[pl.ds(toff, tail)],
                                idx_v.at[pl.ds(0, tail)])
                cps = [pltpu.async_copy(
                    h_hbm.at[idx_v.at[pl.ds(st, ln)]],
                    rows_v.at[pl.ds(st, ln)], sem_g)
                    for (st, ln) in tail_slices]
                for cp in cps:
                    cp.wait()
                pltpu.sync_copy(rows_v.at[pl.ds(0, tail)],
                                cat_hbm.at[pl.ds(toff - e0, tail),
                                           pl.ds(col, _EMB)])
            one(dst_hbm, idx_a, rows_a, 0)
            one(src_hbm, idx_b, rows_b, _EMB)

    return k(h64, src, dst)


def _scatter(wmsg2, dst, e0, n):
    rows_per_s = n // 2 // _NS
    n_full = rows_per_s // 128
    tail = rows_per_s - n_full * 128

    @functools.partial(
        pl.kernel,
        out_type=jax.ShapeDtypeStruct((_HALF, 2 * _EMB), jnp.float32),
        mesh=_sc_mesh(),
        scratch_types=[
            pltpu.VMEM((256,), jnp.int32),
            pltpu.VMEM((2, 128), jnp.int32),
            pltpu.VMEM((128, _EMB), jnp.float32),
            pltpu.VMEM((128, _EMB), jnp.float32),
            pltpu.VMEM_SHARED((_ACC_ROWS, _EMB), jnp.float32),
            pltpu.SemaphoreType.DMA,
            pltpu.SemaphoreType.DMA,
        ],
        compiler_params=pltpu.CompilerParams(use_tc_tiling_on_sc=False),
    )
    def k(w_hbm, d_hbm, out_hbm, raw_v, idx2_v, vals_v, vals2_v, acc,
          sem_r, sem_a):
        c = lax.axis_index("c")
        s = lax.axis_index("s")
        lane = lax.iota(jnp.int32, 16)
        row_base = s * rows_per_s
        nodes0 = c * _HALF

        def fixup(kv, nvalid):
            v = raw_v[pl.ds(kv * 16, 16)]
            local = v - nodes0
            inr = (local >= 0) & (local < _HALF)
            if nvalid < 16:
                inr = inr & (lane < nvalid)
            lrow = (local >> 1) + (local & 1) * _ODD_BASE
            pad = _PAD_BASE + s * 16 + ((lane + kv) & 15)
            idx2_v[kv // 8, pl.ds((kv % 8) * 16, 16)] = (
                jnp.where(inr, lrow, pad))

        def zrow(r, carry):
            for t in range(_EMB // 16):
                vals_v[r, pl.ds(t * 16, 16)] = jnp.zeros((16,), jnp.float32)
            return carry
        lax.fori_loop(0, 128, zrow, 0)
        zb = s * (_ACC_ROWS // _NS)
        for t in range(_ACC_ROWS // _NS // 128):
            pltpu.sync_copy(vals_v, acc.at[pl.ds(zb + t * 128, 128)])
        rem = (_ACC_ROWS // _NS) % 128
        if rem:
            pltpu.sync_copy(
                vals_v.at[pl.ds(0, rem)],
                acc.at[pl.ds(zb + (_ACC_ROWS // _NS) - rem, rem)])
        plsc.subcore_barrier()

        def chunk(roff, nrows):
            cv1 = pltpu.async_copy(
                w_hbm.at[pl.ds(roff, nrows), pl.ds(0, _EMB)],
                vals_v.at[pl.ds(0, nrows)], sem_r)
            cv2 = pltpu.async_copy(
                w_hbm.at[pl.ds(roff, nrows), pl.ds(_EMB, _EMB)],
                vals2_v.at[pl.ds(0, nrows)], sem_r)
            ci1 = pltpu.async_copy(d_hbm.at[pl.ds(e0 + roff, nrows)],
                                   raw_v.at[pl.ds(0, nrows)], sem_r)
            ci2 = pltpu.async_copy(
                d_hbm.at[pl.ds(e0 + n // 2 + roff, nrows)],
                raw_v.at[pl.ds(128, nrows)], sem_r)
            ci1.wait()
            ci2.wait()
            for half in range(2):
                for kv in range(8):
                    nvalid = max(0, min(16, nrows - kv * 16))
                    fixup(half * 8 + kv, nvalid)
            cv1.wait()
            cv2.wait()
            a1 = pltpu.async_copy(vals_v, acc.at[idx2_v.at[0]], sem_a,
                                  add=True)
            a1.wait()
            a2 = pltpu.async_copy(vals2_v, acc.at[idx2_v.at[1]], sem_a,
                                  add=True)
            a2.wait()

        def step(i, carry):
            chunk(row_base + i * 128, 128)
            return carry

        lax.fori_loop(0, n_full, step, 0)
        if tail:
            chunk(row_base + n_full * 128, tail)
        plsc.subcore_barrier()

        rows = _QUART // 4
        @pl.when(s < 4)
        def _():
            pltpu.sync_copy(
                acc.at[pl.ds(s * rows, rows)],
                out_hbm.at[pl.ds(c * _QUART + s * rows, rows),
                           pl.ds(0, _EMB)])

        @pl.when((s >= 4) & (s < 8))
        def _():
            pltpu.sync_copy(
                acc.at[pl.ds(_ODD_BASE + (s - 4) * rows, rows)],
                out_hbm.at[pl.ds(c * _QUART + (s - 4) * rows, rows),
                           pl.ds(_EMB, _EMB)])

    return k(wmsg2, dst)


def _full_spec(d1, d2):
    return pl.BlockSpec((d1, d2), lambda i: (0, 0))


def _row_spec(blk, d):
    return pl.BlockSpec((blk, d), lambda i: (i, 0))


def _lnorm(x, eps=1e-5):
    m = jnp.mean(x, axis=-1, keepdims=True)
    v = jnp.mean((x - m) ** 2, axis=-1, keepdims=True)
    return (x - m) * lax.rsqrt(v + eps)


def _bd(w):
    z = jnp.zeros_like(w)
    return jnp.concatenate(
        [jnp.concatenate([w, z], 1), jnp.concatenate([z, w], 1)], 0)


def _dup(b):
    return jnp.concatenate([b, b]).reshape(1, -1)


def _enc_body(oe, oo, w1, b1, g1, be1, w2, b2, out):
    def half(o):
        h = jnp.dot(o[...], w1[...], preferred_element_type=jnp.float32)
        h = _lnorm(h + b1[...]) * g1[...] + be1[...]
        h = jnp.maximum(h, 0.0)
        h2 = jnp.dot(h, w2[...], preferred_element_type=jnp.float32)
        return jnp.maximum(h2 + b2[...], 0.0)
    out[...] = jnp.concatenate([half(oe), half(oo)], axis=-1)


def _encoder(obs_e, obs_o, p):
    return pl.pallas_call(
        _enc_body,
        grid=(_HALF // _BLK_N,),
        in_specs=[
            _row_spec(_BLK_N, 10), _row_spec(_BLK_N, 10),
            _full_spec(10, _HID), _full_spec(1, _HID),
            _full_spec(1, _HID), _full_spec(1, _HID),
            _full_spec(_HID, _EMB), _full_spec(1, _EMB),
        ],
        out_specs=_row_spec(_BLK_N, 2 * _EMB),
        out_shape=jax.ShapeDtypeStruct((_HALF, 2 * _EMB), jnp.float32),
    )(obs_e, obs_o, p['enc_w1'], p['enc_b1'].reshape(1, -1),
      p['enc_g1'].reshape(1, -1), p['enc_be1'].reshape(1, -1),
      p['enc_w2'], p['enc_b2'].reshape(1, -1))


def _edge_body(cat1, cat2, eaT1, eaT2, wij, we, b1, w2, b2, out):
    def part(cat, eaT):
        pre = jnp.dot(cat[...], wij[...], preferred_element_type=jnp.float32)
        pre = pre + lax.dot_general(
            eaT[...], we[...], (((0,), (0,)), ((), ())),
            preferred_element_type=jnp.float32)
        h1 = jnp.maximum(pre + b1[...], 0.0)
        msg = jnp.dot(h1, w2[...], preferred_element_type=jnp.float32)
        return 0.25 * (msg + b2[...])
    out[...] = jnp.concatenate([part(cat1, eaT1), part(cat2, eaT2)], -1)


def _edge_mlp(cat, eaT, e0, n, wij, we, b1, w2, b2):
    hb = n // 2 // _BLK_E
    ea1 = e0 // _BLK_E
    ea2 = (e0 + n // 2) // _BLK_E
    return pl.pallas_call(
        _edge_body,
        grid=(hb,),
        in_specs=[
            pl.BlockSpec((_BLK_E, 2 * _EMB), lambda i: (i, 0)),
            pl.BlockSpec((_BLK_E, 2 * _EMB), lambda i, _hb=hb: (i + _hb, 0)),
            pl.BlockSpec((_EDGE, _BLK_E), lambda i, _o=ea1: (0, i + _o)),
            pl.BlockSpec((_EDGE, _BLK_E), lambda i, _o=ea2: (0, i + _o)),
            _full_spec(2 * _EMB, _HID), _full_spec(_EDGE, _HID),
            _full_spec(1, _HID),
            _full_spec(_HID, _EMB), _full_spec(1, _EMB),
        ],
        out_specs=_row_spec(_BLK_E, 2 * _EMB),
        out_shape=jax.ShapeDtypeStruct((n // 2, 2 * _EMB), jnp.float32),
    )(cat, cat, eaT, eaT, wij, we, b1.reshape(1, -1), w2, b2.reshape(1, -1))


def _upd_body(h, aa, ab, w1h, w1a, b1, w2, b2, g, b, out):
    u = jnp.dot(h[...], w1h[...], preferred_element_type=jnp.float32)
    u = u + jnp.dot(aa[...] + ab[...], w1a[...],
                    preferred_element_type=jnp.float32)
    u = jnp.maximum(u + b1[...], 0.0)
    upd = jnp.dot(u, w2[...], preferred_element_type=jnp.float32) + b2[...]
    y = h[...] + upd
    yl = jnp.concatenate(
        [_lnorm(y[:, :_EMB]), _lnorm(y[:, _EMB:])], axis=-1)
    out[...] = yl * g[...] + b[...]


def _update(h_p, aggr_a, aggr_b, bw1h, bw1a, b1, bw2, b2, g, b):
    return pl.pallas_call(
        _upd_body,
        grid=(_HALF // _BLK_N,),
        in_specs=[
            _row_spec(_BLK_N, 2 * _EMB), _row_spec(_BLK_N, 2 * _EMB),
            _row_spec(_BLK_N, 2 * _EMB),
            _full_spec(2 * _EMB, 2 * _HID), _full_spec(2 * _EMB, 2 * _HID),
            _full_spec(1, 2 * _HID),
            _full_spec(2 * _HID, 2 * _EMB), _full_spec(1, 2 * _EMB),
            _full_spec(1, 2 * _EMB), _full_spec(1, 2 * _EMB),
        ],
        out_specs=_row_spec(_BLK_N, 2 * _EMB),
        out_shape=jax.ShapeDtypeStruct((_HALF, 2 * _EMB), jnp.float32),
    )(h_p, aggr_a, aggr_b, bw1h, bw1a, b1, bw2, b2, g, b)


def _out_body(h, w1, b1, w2, b2, out):
    u = jnp.dot(h[...], w1[...], preferred_element_type=jnp.float32)
    u = jnp.maximum(u + b1[...], 0.0)
    out[...] = jnp.dot(u, w2[...], preferred_element_type=jnp.float32) + b2[...]


def _output(h_p, bw1, b1, bw2, b2):
    return pl.pallas_call(
        _out_body,
        grid=(_HALF // _BLK_N,),
        in_specs=[
            _row_spec(_BLK_N, 2 * _EMB),
            _full_spec(2 * _EMB, 2 * _HID), _full_spec(1, 2 * _HID),
            _full_spec(2 * _HID, 2 * _EMB), _full_spec(1, 2 * _EMB),
        ],
        out_specs=_row_spec(_BLK_N, 2 * _EMB),
        out_shape=jax.ShapeDtypeStruct((_HALF, 2 * _EMB), jnp.float32),
    )(h_p, bw1, b1, bw2, b2)


def kernel(obs, edge_index, edge_attr, params):
    p = params
    src = edge_index[0]
    dst = edge_index[1]
    obs_e = obs[0::2, :10]
    obs_o = obs[1::2, :10]
    eaT = edge_attr.T

    h_p = _encoder(obs_e, obs_o, p)
    for lp in p['layers']:
        h64 = h_p.reshape(_N, _EMB)
        w1 = lp['msg_w1']
        cat_a = _gather(h64, src, dst, 0, _EA)
        cat_b = _gather(h64, src, dst, _EA, _EB)
        wmsg_a = _edge_mlp(cat_a, eaT, 0, _EA, w1[:2 * _EMB], w1[2 * _EMB:],
                           lp['msg_b1'], lp['msg_w2'], lp['msg_b2'])
        aggr_a = _scatter(wmsg_a, dst, 0, _EA)
        wmsg_b = _edge_mlp(cat_b, eaT, _EA, _EB, w1[:2 * _EMB],
                           w1[2 * _EMB:], lp['msg_b1'], lp['msg_w2'],
                           lp['msg_b2'])
        aggr_b = _scatter(wmsg_b, dst, _EA, _EB)
        uw1 = lp['upd_w1']
        h_p = _update(h_p, aggr_a, aggr_b,
                      _bd(uw1[:_EMB]), _bd(uw1[_EMB:]),
                      _dup(lp['upd_b1']), _bd(lp['upd_w2']),
                      _dup(lp['upd_b2']), _dup(lp['ln_g']),
                      _dup(lp['ln_b']))
    out_p = _output(h_p, _bd(p['out_w1']), _dup(p['out_b1']),
                    _bd(p['out_w2']), _dup(p['out_b2']))
    return out_p.reshape(_N, _EMB)


# speedup vs baseline: 1.0001x; 1.0001x over previous
import functools

import jax
import jax.numpy as jnp
from jax import lax
from jax.experimental import pallas as pl
from jax.experimental.pallas import tpu as pltpu
from jax.experimental.pallas import tpu_sc as plsc

_N = 50000
_E = 800000
_EMB = 64
_HID = 128
_EDGE = 8

_NC = 2
_NS = 16
_NW = _NC * _NS

_EA = 409600
_EB = _E - _EA

_GCHUNK = 512

_HALF = _N // 2
_QUART = _HALF // 2
_ODD_BASE = 12800
_PAD_BASE = 25300
_ACC_ROWS = 25600

_BLK_N = 5000
_BLK_E = 3200


def _sc_mesh():
    return plsc.VectorSubcoreMesh(
        core_axis_name="c", subcore_axis_name="s",
        num_cores=_NC, num_subcores=_NS)


def _gather(h64, src, dst, e0, n):
    per_w = n // _NW
    n_full = per_w // _GCHUNK
    tail = per_w - n_full * _GCHUNK

    @functools.partial(
        pl.kernel,
        out_type=jax.ShapeDtypeStruct((n, 2 * _EMB), jnp.float32),
        mesh=_sc_mesh(),
        scratch_types=[
            pltpu.VMEM((_GCHUNK,), jnp.int32),
            pltpu.VMEM((_GCHUNK,), jnp.int32),
            pltpu.VMEM((_GCHUNK, _EMB), jnp.float32),
            pltpu.VMEM((_GCHUNK, _EMB), jnp.float32),
            pltpu.SemaphoreType.DMA,
            pltpu.SemaphoreType.DMA,
            pltpu.SemaphoreType.DMA,
        ],
        compiler_params=pltpu.CompilerParams(use_tc_tiling_on_sc=False),
    )
    def k(h_hbm, src_hbm, dst_hbm, cat_hbm,
          idx_a, idx_b, rows_a, rows_b, sem_i, sem_g, sem_o):
        c = lax.axis_index("c")
        s = lax.axis_index("s")
        wid = s * _NC + c
        base = wid * per_w

        full_slices = [(j * 128, 128) for j in range(_GCHUNK // 128)]
        tail_slices = [(j * 128, 128) for j in range(tail // 128)]
        if tail % 128:
            tail_slices.append((tail - tail % 128, tail % 128))

        def out_copy(rows_v, off, col, m):
            return (rows_v.at[pl.ds(0, m)],
                    cat_hbm.at[pl.ds(off - e0, m), pl.ds(col, _EMB)])

        def step(i, carry):
            off = e0 + base + i * _GCHUNK

            @pl.when(i > 0)
            def _():
                sa, da = out_copy(rows_a, off, 0, _GCHUNK)
                pltpu.make_async_copy(sa, da, sem_o).wait()
                sb, db = out_copy(rows_b, off, _EMB, _GCHUNK)
                pltpu.make_async_copy(sb, db, sem_o).wait()

            ca = pltpu.async_copy(dst_hbm.at[pl.ds(off, _GCHUNK)],
                                  idx_a, sem_i)
            cb = pltpu.async_copy(src_hbm.at[pl.ds(off, _GCHUNK)],
                                  idx_b, sem_i)
            ca.wait()
            cb.wait()
            cps = [pltpu.async_copy(
                h_hbm.at[idx_a.at[pl.ds(st, ln)]],
                rows_a.at[pl.ds(st, ln)], sem_g)
                for (st, ln) in full_slices]
            cps += [pltpu.async_copy(
                h_hbm.at[idx_b.at[pl.ds(st, ln)]],
                rows_b.at[pl.ds(st, ln)], sem_g)
                for (st, ln) in full_slices]
            for cp in cps:
                cp.wait()
            sa, da = out_copy(rows_a, off, 0, _GCHUNK)
            pltpu.async_copy(sa, da, sem_o)
            sb, db = out_copy(rows_b, off, _EMB, _GCHUNK)
            pltpu.async_copy(sb, db, sem_o)
            return carry

        lax.fori_loop(0, n_full, step, 0)
        sa, da = out_copy(rows_a, e0 + base, 0, _GCHUNK)
        pltpu.make_async_copy(sa, da, sem_o).wait()
        sb, db = out_copy(rows_b, e0 + base, _EMB, _GCHUNK)
        pltpu.make_async_copy(sb, db, sem_o).wait()

        if tail:
            toff = e0 + base + n_full * _GCHUNK

            def one(idx_hbm, idx_v, rows_v, col):
                pltpu.sync_copy(idx_hbm.at[pl.ds(toff, tail)],
                                idx_v.at[pl.ds(0, tail)])
                cps = [pltpu.async_copy(
                    h_hbm.at[idx_v.at[pl.ds(st, ln)]],
                    rows_v.at[pl.ds(st, ln)], sem_g)
                    for (st, ln) in tail_slices]
                for cp in cps:
                    cp.wait()
                pltpu.sync_copy(rows_v.at[pl.ds(0, tail)],
                                cat_hbm.at[pl.ds(toff - e0, tail),
                                           pl.ds(col, _EMB)])
            one(dst_hbm, idx_a, rows_a, 0)
            one(src_hbm, idx_b, rows_b, _EMB)

    return k(h64, src, dst)


def _scatter(wmsg2, dst, e0, n):
    rows_per_s = n // 2 // _NS
    n_full = rows_per_s // 128
    tail = rows_per_s - n_full * 128

    @functools.partial(
        pl.kernel,
        out_type=jax.ShapeDtypeStruct((_HALF, 2 * _EMB), jnp.float32),
        mesh=_sc_mesh(),
        scratch_types=[
            pltpu.VMEM((256,), jnp.int32),
            pltpu.VMEM((2, 128), jnp.int32),
            pltpu.VMEM((128, _EMB), jnp.float32),
            pltpu.VMEM((128, _EMB), jnp.float32),
            pltpu.VMEM_SHARED((_ACC_ROWS, _EMB), jnp.float32),
            pltpu.SemaphoreType.DMA,
            pltpu.SemaphoreType.DMA,
            pltpu.SemaphoreType.DMA,
        ],
        compiler_params=pltpu.CompilerParams(use_tc_tiling_on_sc=False),
    )
    def k(w_hbm, d_hbm, out_hbm, raw_v, idx2_v, vals_v, vals2_v, acc,
          sem_r, sem_d, sem_a):
        c = lax.axis_index("c")
        s = lax.axis_index("s")
        lane = lax.iota(jnp.int32, 16)
        row_base = s * rows_per_s
        nodes0 = c * _HALF

        def fixup(kv, nvalid):
            v = raw_v[pl.ds(kv * 16, 16)]
            local = v - nodes0
            inr = (local >= 0) & (local < _HALF)
            if nvalid < 16:
                inr = inr & (lane < nvalid)
            lrow = (local >> 1) + (local & 1) * _ODD_BASE
            pad = _PAD_BASE + s * 16 + ((lane + kv) & 15)
            idx2_v[kv // 8, pl.ds((kv % 8) * 16, 16)] = (
                jnp.where(inr, lrow, pad))

        def zrow(r, carry):
            for t in range(_EMB // 16):
                vals_v[r, pl.ds(t * 16, 16)] = jnp.zeros((16,), jnp.float32)
            return carry
        lax.fori_loop(0, 128, zrow, 0)
        zb = s * (_ACC_ROWS // _NS)
        for t in range(_ACC_ROWS // _NS // 128):
            pltpu.sync_copy(vals_v, acc.at[pl.ds(zb + t * 128, 128)])
        rem = (_ACC_ROWS // _NS) % 128
        if rem:
            pltpu.sync_copy(
                vals_v.at[pl.ds(0, rem)],
                acc.at[pl.ds(zb + (_ACC_ROWS // _NS) - rem, rem)])
        plsc.subcore_barrier()

        def chunk(roff, nrows):
            cv1 = pltpu.async_copy(
                w_hbm.at[pl.ds(roff, nrows), pl.ds(0, _EMB)],
                vals_v.at[pl.ds(0, nrows)], sem_r)
            cv2 = pltpu.async_copy(
                w_hbm.at[pl.ds(roff, nrows), pl.ds(_EMB, _EMB)],
                vals2_v.at[pl.ds(0, nrows)], sem_r)
            ci1 = pltpu.async_copy(d_hbm.at[pl.ds(e0 + roff, nrows)],
                                   raw_v.at[pl.ds(0, nrows)], sem_d)
            ci2 = pltpu.async_copy(
                d_hbm.at[pl.ds(e0 + n // 2 + roff, nrows)],
                raw_v.at[pl.ds(128, nrows)], sem_d)
            ci1.wait()
            ci2.wait()
            for half in range(2):
                for kv in range(8):
                    nvalid = max(0, min(16, nrows - kv * 16))
                    fixup(half * 8 + kv, nvalid)
            cv1.wait()
            cv2.wait()
            a1 = pltpu.async_copy(vals_v, acc.at[idx2_v.at[0]], sem_a,
                                  add=True)
            a1.wait()
            a2 = pltpu.async_copy(vals2_v, acc.at[idx2_v.at[1]], sem_a,
                                  add=True)
            a2.wait()

        def step(i, carry):
            chunk(row_base + i * 128, 128)
            return carry

        lax.fori_loop(0, n_full, step, 0)
        if tail:
            chunk(row_base + n_full * 128, tail)
        plsc.subcore_barrier()

        rows = _QUART // 4
        @pl.when(s < 4)
        def _():
            pltpu.sync_copy(
                acc.at[pl.ds(s * rows, rows)],
                out_hbm.at[pl.ds(c * _QUART + s * rows, rows),
                           pl.ds(0, _EMB)])

        @pl.when((s >= 4) & (s < 8))
        def _():
            pltpu.sync_copy(
                acc.at[pl.ds(_ODD_BASE + (s - 4) * rows, rows)],
                out_hbm.at[pl.ds(c * _QUART + (s - 4) * rows, rows),
                           pl.ds(_EMB, _EMB)])

    return k(wmsg2, dst)


def _full_spec(d1, d2):
    return pl.BlockSpec((d1, d2), lambda i: (0, 0))


def _row_spec(blk, d):
    return pl.BlockSpec((blk, d), lambda i: (i, 0))


def _lnorm(x, eps=1e-5):
    m = jnp.mean(x, axis=-1, keepdims=True)
    v = jnp.mean((x - m) ** 2, axis=-1, keepdims=True)
    return (x - m) * lax.rsqrt(v + eps)


def _bd(w):
    z = jnp.zeros_like(w)
    return jnp.concatenate(
        [jnp.concatenate([w, z], 1), jnp.concatenate([z, w], 1)], 0)


def _dup(b):
    return jnp.concatenate([b, b]).reshape(1, -1)


def _enc_body(oe, oo, w1, b1, g1, be1, w2, b2, out):
    def half(o):
        h = jnp.dot(o[...], w1[...], preferred_element_type=jnp.float32)
        h = _lnorm(h + b1[...]) * g1[...] + be1[...]
        h = jnp.maximum(h, 0.0)
        h2 = jnp.dot(h, w2[...], preferred_element_type=jnp.float32)
        return jnp.maximum(h2 + b2[...], 0.0)
    out[...] = jnp.concatenate([half(oe), half(oo)], axis=-1)


def _encoder(obs_e, obs_o, p):
    return pl.pallas_call(
        _enc_body,
        grid=(_HALF // _BLK_N,),
        in_specs=[
            _row_spec(_BLK_N, 10), _row_spec(_BLK_N, 10),
            _full_spec(10, _HID), _full_spec(1, _HID),
            _full_spec(1, _HID), _full_spec(1, _HID),
            _full_spec(_HID, _EMB), _full_spec(1, _EMB),
        ],
        out_specs=_row_spec(_BLK_N, 2 * _EMB),
        out_shape=jax.ShapeDtypeStruct((_HALF, 2 * _EMB), jnp.float32),
    )(obs_e, obs_o, p['enc_w1'], p['enc_b1'].reshape(1, -1),
      p['enc_g1'].reshape(1, -1), p['enc_be1'].reshape(1, -1),
      p['enc_w2'], p['enc_b2'].reshape(1, -1))


def _edge_body(cat1, cat2, eaT1, eaT2, wij, we, b1, w2, b2, out):
    def part(cat, eaT):
        pre = jnp.dot(cat[...], wij[...], preferred_element_type=jnp.float32)
        pre = pre + lax.dot_general(
            eaT[...], we[...], (((0,), (0,)), ((), ())),
            preferred_element_type=jnp.float32)
        h1 = jnp.maximum(pre + b1[...], 0.0)
        msg = jnp.dot(h1, w2[...], preferred_element_type=jnp.float32)
        return 0.25 * (msg + b2[...])
    out[...] = jnp.concatenate([part(cat1, eaT1), part(cat2, eaT2)], -1)


def _edge_mlp(cat, eaT, e0, n, wij, we, b1, w2, b2):
    hb = n // 2 // _BLK_E
    ea1 = e0 // _BLK_E
    ea2 = (e0 + n // 2) // _BLK_E
    return pl.pallas_call(
        _edge_body,
        grid=(hb,),
        in_specs=[
            pl.BlockSpec((_BLK_E, 2 * _EMB), lambda i: (i, 0)),
            pl.BlockSpec((_BLK_E, 2 * _EMB), lambda i, _hb=hb: (i + _hb, 0)),
            pl.BlockSpec((_EDGE, _BLK_E), lambda i, _o=ea1: (0, i + _o)),
            pl.BlockSpec((_EDGE, _BLK_E), lambda i, _o=ea2: (0, i + _o)),
            _full_spec(2 * _EMB, _HID), _full_spec(_EDGE, _HID),
            _full_spec(1, _HID),
            _full_spec(_HID, _EMB), _full_spec(1, _EMB),
        ],
        out_specs=_row_spec(_BLK_E, 2 * _EMB),
        out_shape=jax.ShapeDtypeStruct((n // 2, 2 * _EMB), jnp.float32),
    )(cat, cat, eaT, eaT, wij, we, b1.reshape(1, -1), w2, b2.reshape(1, -1))


def _upd_body(h, aa, ab, w1h, w1a, b1, w2, b2, g, b, out):
    u = jnp.dot(h[...], w1h[...], preferred_element_type=jnp.float32)
    u = u + jnp.dot(aa[...] + ab[...], w1a[...],
                    preferred_element_type=jnp.float32)
    u = jnp.maximum(u + b1[...], 0.0)
    upd = jnp.dot(u, w2[...], preferred_element_type=jnp.float32) + b2[...]
    y = h[...] + upd
    yl = jnp.concatenate(
        [_lnorm(y[:, :_EMB]), _lnorm(y[:, _EMB:])], axis=-1)
    out[...] = yl * g[...] + b[...]


def _update(h_p, aggr_a, aggr_b, bw1h, bw1a, b1, bw2, b2, g, b):
    return pl.pallas_call(
        _upd_body,
        grid=(_HALF // _BLK_N,),
        in_specs=[
            _row_spec(_BLK_N, 2 * _EMB), _row_spec(_BLK_N, 2 * _EMB),
            _row_spec(_BLK_N, 2 * _EMB),
            _full_spec(2 * _EMB, 2 * _HID), _full_spec(2 * _EMB, 2 * _HID),
            _full_spec(1, 2 * _HID),
            _full_spec(2 * _HID, 2 * _EMB), _full_spec(1, 2 * _EMB),
            _full_spec(1, 2 * _EMB), _full_spec(1, 2 * _EMB),
        ],
        out_specs=_row_spec(_BLK_N, 2 * _EMB),
        out_shape=jax.ShapeDtypeStruct((_HALF, 2 * _EMB), jnp.float32),
    )(h_p, aggr_a, aggr_b, bw1h, bw1a, b1, bw2, b2, g, b)


def _out_body(h, w1, b1, w2, b2, out):
    u = jnp.dot(h[...], w1[...], preferred_element_type=jnp.float32)
    u = jnp.maximum(u + b1[...], 0.0)
    out[...] = jnp.dot(u, w2[...], preferred_element_type=jnp.float32) + b2[...]


def _output(h_p, bw1, b1, bw2, b2):
    return pl.pallas_call(
        _out_body,
        grid=(_HALF // _BLK_N,),
        in_specs=[
            _row_spec(_BLK_N, 2 * _EMB),
            _full_spec(2 * _EMB, 2 * _HID), _full_spec(1, 2 * _HID),
            _full_spec(2 * _HID, 2 * _EMB), _full_spec(1, 2 * _EMB),
        ],
        out_specs=_row_spec(_BLK_N, 2 * _EMB),
        out_shape=jax.ShapeDtypeStruct((_HALF, 2 * _EMB), jnp.float32),
    )(h_p, bw1, b1, bw2, b2)


def kernel(obs, edge_index, edge_attr, params):
    p = params
    src = edge_index[0]
    dst = edge_index[1]
    obs_e = obs[0::2, :10]
    obs_o = obs[1::2, :10]
    eaT = edge_attr.T

    h_p = _encoder(obs_e, obs_o, p)
    for lp in p['layers']:
        h64 = h_p.reshape(_N, _EMB)
        w1 = lp['msg_w1']
        cat_a = _gather(h64, src, dst, 0, _EA)
        cat_b = _gather(h64, src, dst, _EA, _EB)
        wmsg_a = _edge_mlp(cat_a, eaT, 0, _EA, w1[:2 * _EMB], w1[2 * _EMB:],
                           lp['msg_b1'], lp['msg_w2'], lp['msg_b2'])
        aggr_a = _scatter(wmsg_a, dst, 0, _EA)
        wmsg_b = _edge_mlp(cat_b, eaT, _EA, _EB, w1[:2 * _EMB],
                           w1[2 * _EMB:], lp['msg_b1'], lp['msg_w2'],
                           lp['msg_b2'])
        aggr_b = _scatter(wmsg_b, dst, _EA, _EB)
        uw1 = lp['upd_w1']
        h_p = _update(h_p, aggr_a, aggr_b,
                      _bd(uw1[:_EMB]), _bd(uw1[_EMB:]),
                      _dup(lp['upd_b1']), _bd(lp['upd_w2']),
                      _dup(lp['upd_b2']), _dup(lp['ln_g']),
                      _dup(lp['ln_b']))
    out_p = _output(h_p, _bd(p['out_w1']), _dup(p['out_b1']),
                    _bd(p['out_w2']), _dup(p['out_b2']))
    return out_p.reshape(_N, _EMB)
